# baseline (speedup 1.0000x reference)
import jax
import jax.numpy as jnp
from jax.experimental import pallas as pl

B = 1024
N = 256
FIN = 7
H1 = 32
H2 = 16
H3 = 128
LAT = 64
NOC = 16
HOC = 8
HPC = 8
GDH = 16
BBXD = 6
LBLD = 1

BBLK = 16
SUB = 4
NG = B // BBLK
BBLK2 = 256


def _gcn_body(E_ref, Xp_ref, w1_ref, b1_ref, w2_ref, b2_ref, i_ref, h2t_ref):
    f32 = jnp.float32
    bf16 = jnp.bfloat16
    w1 = w1_ref[...]
    b1 = b1_ref[...]
    w2 = w2_ref[...]
    b2 = b2_ref[...]
    ident = i_ref[...]
    tdims = (((0,), (0,)), ((), ()))
    for g in range(BBLK // SUB):
        E2 = E_ref[g * SUB:(g + 1) * SUB].reshape(SUB * N, N).astype(bf16)
        xp = Xp_ref[0][:, g * SUB * FIN:(g + 1) * SUB * FIN]
        t1 = jnp.dot(E2, xp.astype(bf16), preferred_element_type=f32)
        t1d = jnp.concatenate(
            [t1[b * N:(b + 1) * N, b * FIN:(b + 1) * FIN] for b in range(SUB)],
            axis=0)
        h1 = jnp.maximum(jnp.dot(t1d, w1, preferred_element_type=f32) + b1, 0.0)
        h1c = jnp.concatenate(
            [h1[b * N:(b + 1) * N] for b in range(SUB)], axis=1)
        t2 = jnp.dot(E2, h1c.astype(bf16), preferred_element_type=f32)
        t2d = jnp.concatenate(
            [t2[b * N:(b + 1) * N, b * H1:(b + 1) * H1] for b in range(SUB)],
            axis=0)
        h2 = jnp.maximum(jnp.dot(t2d, w2, preferred_element_type=f32) + b2, 0.0)
        for b in range(SUB):
            piece = h2[b * N:(b + 1) * N]
            h2t_ref[g * SUB + b] = jax.lax.dot_general(
                piece, ident, tdims, preferred_element_type=f32)


def _mlp_body(h2f_ref, oc_ref, xo_ref, nd_ref,
              encWa_ref, encWb_ref, encb_ref,
              zmW_ref, zmb_ref, zlW_ref, zlb_ref,
              de1W_ref, de1b_ref, de2W_ref, de2b_ref, de3W_ref, de3b_ref,
              objW_ref, objb_ref, partW_ref, partb_ref,
              dd1Wa_ref, dd1Wb_ref, dd1Wc_ref, dd1b_ref,
              dd2W_ref, dd2b_ref, dd3W_ref, dd3b_ref,
              gd1Wa_ref, gd1Wb_ref, gd1Wc_ref, gd1Wd_ref, gd1b_ref,
              gd2W_ref, gd2b_ref, bdb_ref, bdl_ref, bbn_ref, bln_ref,
              zm_ref, zl_ref, xob_ref, xbn_ref, xl_ref):
    f32 = jnp.float32
    bf16 = jnp.bfloat16
    h2f = h2f_ref[...]
    oc_raw = oc_ref[...]
    h3 = jnp.maximum(
        jnp.dot(h2f, encWa_ref[...], preferred_element_type=f32)
        + jnp.dot(oc_raw, encWb_ref[...], preferred_element_type=f32)
        + encb_ref[...], 0.0)
    z_mean = jnp.dot(h3, zmW_ref[...], preferred_element_type=f32) + zmb_ref[...]
    z_logvar = jnp.dot(h3, zlW_ref[...], preferred_element_type=f32) + zlb_ref[...]
    zm_ref[...] = z_mean
    zl_ref[...] = z_logvar
    lo = jnp.maximum(jnp.dot(xo_ref[...], de1W_ref[...], preferred_element_type=f32) + de1b_ref[...], 0.0)
    lo = jnp.maximum(jnp.dot(lo, de2W_ref[...], preferred_element_type=f32) + de2b_ref[...], 0.0)
    latent_obj = jnp.dot(lo, de3W_ref[...], preferred_element_type=f32) + de3b_ref[...]
    oc = jnp.dot(oc_raw, objW_ref[...], preferred_element_type=f32) + objb_ref[...]
    nd = jnp.dot(nd_ref[...], partW_ref[...], preferred_element_type=f32) + partb_ref[...]
    d = jnp.maximum(
        jnp.dot(nd, dd1Wa_ref[...], preferred_element_type=f32)
        + jnp.dot(oc, dd1Wb_ref[...], preferred_element_type=f32)
        + jnp.dot(latent_obj, dd1Wc_ref[...], preferred_element_type=f32)
        + dd1b_ref[...], 0.0)
    d = jnp.maximum(jnp.dot(d, dd2W_ref[...], preferred_element_type=f32) + dd2b_ref[...], 0.0)
    xob_ref[...] = jax.nn.sigmoid(jnp.dot(d, dd3W_ref[...], preferred_element_type=f32) + dd3b_ref[...])
    a1 = jnp.maximum(
        jnp.dot(nd, gd1Wa_ref[...], preferred_element_type=f32)
        + jnp.dot(oc, gd1Wb_ref[...], preferred_element_type=f32)
        + jnp.dot(latent_obj, gd1Wc_ref[...], preferred_element_type=f32)
        + jnp.dot(z_mean, gd1Wd_ref[...], preferred_element_type=f32)
        + gd1b_ref[...], 0.0)
    gq = jnp.maximum(jnp.dot(a1, gd2W_ref[...], preferred_element_type=f32) + gd2b_ref[...], 0.0)
    gqb = gq.astype(bf16)
    xbn_ref[...] = jax.nn.sigmoid(
        jnp.dot(gqb, bdb_ref[...], preferred_element_type=f32) + bbn_ref[...])
    xl_ref[...] = jax.nn.sigmoid(
        jnp.dot(gqb, bdl_ref[...], preferred_element_type=f32) + bln_ref[...])


def _full(shape):
    ndim = len(shape)
    return pl.BlockSpec(shape, lambda i, *, _nd=ndim: (0,) * _nd)


def kernel(E, X_part, X_obj, nodes, obj_class, params):
    p = params
    f32 = jnp.float32

    def r2(v):
        return v.reshape(1, -1)

    XC = X_part.reshape(NG, BBLK, N, FIN).transpose(0, 2, 1, 3).reshape(NG, N, BBLK * FIN)
    ident = jnp.eye(N, dtype=f32)

    H2T = pl.pallas_call(
        _gcn_body,
        grid=(NG,),
        in_specs=[
            pl.BlockSpec((BBLK, N, N), lambda i: (i, 0, 0)),
            pl.BlockSpec((1, N, BBLK * FIN), lambda i: (i, 0, 0)),
            _full((FIN, H1)), _full((1, H1)),
            _full((H1, H2)), _full((1, H2)),
            _full((N, N)),
        ],
        out_specs=pl.BlockSpec((BBLK, H2, N), lambda i: (i, 0, 0)),
        out_shape=jax.ShapeDtypeStruct((B, H2, N), f32),
    )(E, XC, p['gc1_W'], r2(p['gc1_b']), p['gc2_W'], r2(p['gc2_b']), ident)

    h2f = H2T.reshape(B, N * H2)
    encW = p['enc_h3_W']
    encWa = encW[: N * H2].reshape(N, H2, H3).transpose(1, 0, 2).reshape(N * H2, H3)

    r = jnp.arange(N * GDH)
    cb = jnp.arange(N * BBXD)
    bdb = (p['bbx_W'][r % GDH][:, cb % BBXD]
           * (r[:, None] // GDH == cb[None, :] // BBXD)).astype(jnp.bfloat16)
    cl = jnp.arange(N)
    bdl = (p['lbl_W'][r % GDH, 0][:, None]
           * (r[:, None] // GDH == cl[None, :])).astype(jnp.bfloat16)
    bbn = jnp.tile(p['bbx_b'], N).reshape(1, N * BBXD)
    bln = jnp.full((1, N), p['lbl_b'][0], f32)

    dd1W = p['dd1_W']
    gd1W = p['gd1_W']
    weights = [
        encWa, encW[N * H2 :], r2(p['enc_h3_b']),
        p['zmean_W'], r2(p['zmean_b']), p['zlogvar_W'], r2(p['zlogvar_b']),
        p['de1_W'], r2(p['de1_b']), p['de2_W'], r2(p['de2_b']), p['de3_W'], r2(p['de3_b']),
        p['objc_W'], r2(p['objc_b']), p['part_W'], r2(p['part_b']),
        dd1W[:HPC], dd1W[HPC : HPC + HOC], dd1W[HPC + HOC :], r2(p['dd1_b']),
        p['dd2_W'], r2(p['dd2_b']), p['dd3_W'], r2(p['dd3_b']),
        gd1W[:HPC], gd1W[HPC : HPC + HOC], gd1W[HPC + HOC : HPC + HOC + LAT],
        gd1W[HPC + HOC + LAT :], r2(p['gd1_b']),
        p['gd2_W'], r2(p['gd2_b']), bdb, bdl, bbn, bln,
    ]
    z_mean, z_logvar, x_obj_bbx, xbn, xl = pl.pallas_call(
        _mlp_body,
        grid=(B // BBLK2,),
        in_specs=[
            pl.BlockSpec((BBLK2, N * H2), lambda i: (i, 0)),
            pl.BlockSpec((BBLK2, NOC), lambda i: (i, 0)),
            pl.BlockSpec((BBLK2, BBXD), lambda i: (i, 0)),
            pl.BlockSpec((BBLK2, N), lambda i: (i, 0)),
        ] + [_full(w.shape) for w in weights],
        out_specs=[
            pl.BlockSpec((BBLK2, LAT), lambda i: (i, 0)),
            pl.BlockSpec((BBLK2, LAT), lambda i: (i, 0)),
            pl.BlockSpec((BBLK2, BBXD), lambda i: (i, 0)),
            pl.BlockSpec((BBLK2, N * BBXD), lambda i: (i, 0)),
            pl.BlockSpec((BBLK2, N), lambda i: (i, 0)),
        ],
        out_shape=[
            jax.ShapeDtypeStruct((B, LAT), f32),
            jax.ShapeDtypeStruct((B, LAT), f32),
            jax.ShapeDtypeStruct((B, BBXD), f32),
            jax.ShapeDtypeStruct((B, N * BBXD), f32),
            jax.ShapeDtypeStruct((B, N), f32),
        ],
    )(h2f, obj_class, X_obj, nodes, *weights)

    x_bbx = xbn.reshape(B, N, BBXD)
    x_lbl = xl.reshape(B, N, LBLD)
    return (x_bbx, x_obj_bbx, x_lbl, z_mean, z_logvar)


# speedup vs baseline: 1.5898x; 1.5898x over previous
import jax
import jax.numpy as jnp
from jax.experimental import pallas as pl

B = 1024
N = 256
FIN = 7
H1 = 32
H2 = 16
H3 = 128
LAT = 64
NOC = 16
HOC = 8
HPC = 8
GDH = 16
BBXD = 6
LBLD = 1

BBLK = 16
WAVE = 4
NG = B // BBLK
BBLK2 = 256


def _gcn_body(E_ref, Xt_ref, w1t_ref, b1_ref, w2t_ref, b2_ref, h2t_ref):
    f32 = jnp.float32
    bf16 = jnp.bfloat16
    w1t = w1t_ref[...]
    b1 = b1_ref[...]
    w2t = w2t_ref[...]
    b2 = b2_ref[...]
    cdims = (((1,), (1,)), ((), ()))
    for w in range(BBLK // WAVE):
        bs = range(w * WAVE, (w + 1) * WAVE)
        t1s = [jax.lax.dot_general(Xt_ref[b].astype(bf16),
                                   E_ref[b].astype(bf16), cdims,
                                   preferred_element_type=f32) for b in bs]
        h1s = [jnp.maximum(jnp.dot(w1t, t1t, preferred_element_type=f32)
                           + b1, 0.0).astype(bf16) for t1t in t1s]
        t2s = [jax.lax.dot_general(h1s[k], E_ref[b].astype(bf16), cdims,
                                   preferred_element_type=f32)
               for k, b in enumerate(bs)]
        for k, b in enumerate(bs):
            h2t_ref[b] = jnp.maximum(
                jnp.dot(w2t, t2s[k], preferred_element_type=f32) + b2, 0.0)


def _mlp_body(h2f_ref, oc_ref, xo_ref, nd_ref,
              encWa_ref, encWb_ref, encb_ref,
              zmW_ref, zmb_ref, zlW_ref, zlb_ref,
              de1W_ref, de1b_ref, de2W_ref, de2b_ref, de3W_ref, de3b_ref,
              objW_ref, objb_ref, partW_ref, partb_ref,
              dd1Wa_ref, dd1Wb_ref, dd1Wc_ref, dd1b_ref,
              dd2W_ref, dd2b_ref, dd3W_ref, dd3b_ref,
              gd1Wa_ref, gd1Wb_ref, gd1Wc_ref, gd1Wd_ref, gd1b_ref,
              gd2W_ref, gd2b_ref, bdb_ref, bdl_ref, bbn_ref, bln_ref,
              zm_ref, zl_ref, xob_ref, xbn_ref, xl_ref):
    f32 = jnp.float32
    bf16 = jnp.bfloat16
    h2f = h2f_ref[...]
    oc_raw = oc_ref[...]
    h3 = jnp.maximum(
        jnp.dot(h2f, encWa_ref[...], preferred_element_type=f32)
        + jnp.dot(oc_raw, encWb_ref[...], preferred_element_type=f32)
        + encb_ref[...], 0.0)
    z_mean = jnp.dot(h3, zmW_ref[...], preferred_element_type=f32) + zmb_ref[...]
    z_logvar = jnp.dot(h3, zlW_ref[...], preferred_element_type=f32) + zlb_ref[...]
    zm_ref[...] = z_mean
    zl_ref[...] = z_logvar
    lo = jnp.maximum(jnp.dot(xo_ref[...], de1W_ref[...], preferred_element_type=f32) + de1b_ref[...], 0.0)
    lo = jnp.maximum(jnp.dot(lo, de2W_ref[...], preferred_element_type=f32) + de2b_ref[...], 0.0)
    latent_obj = jnp.dot(lo, de3W_ref[...], preferred_element_type=f32) + de3b_ref[...]
    oc = jnp.dot(oc_raw, objW_ref[...], preferred_element_type=f32) + objb_ref[...]
    nd = jnp.dot(nd_ref[...], partW_ref[...], preferred_element_type=f32) + partb_ref[...]
    d = jnp.maximum(
        jnp.dot(nd, dd1Wa_ref[...], preferred_element_type=f32)
        + jnp.dot(oc, dd1Wb_ref[...], preferred_element_type=f32)
        + jnp.dot(latent_obj, dd1Wc_ref[...], preferred_element_type=f32)
        + dd1b_ref[...], 0.0)
    d = jnp.maximum(jnp.dot(d, dd2W_ref[...], preferred_element_type=f32) + dd2b_ref[...], 0.0)
    xob_ref[...] = jax.nn.sigmoid(jnp.dot(d, dd3W_ref[...], preferred_element_type=f32) + dd3b_ref[...])
    a1 = jnp.maximum(
        jnp.dot(nd, gd1Wa_ref[...], preferred_element_type=f32)
        + jnp.dot(oc, gd1Wb_ref[...], preferred_element_type=f32)
        + jnp.dot(latent_obj, gd1Wc_ref[...], preferred_element_type=f32)
        + jnp.dot(z_mean, gd1Wd_ref[...], preferred_element_type=f32)
        + gd1b_ref[...], 0.0)
    gq = jnp.maximum(jnp.dot(a1, gd2W_ref[...], preferred_element_type=f32) + gd2b_ref[...], 0.0)
    gqb = gq.astype(bf16)
    xbn_ref[...] = jax.nn.sigmoid(
        jnp.dot(gqb, bdb_ref[...], preferred_element_type=f32) + bbn_ref[...])
    xl_ref[...] = jax.nn.sigmoid(
        jnp.dot(gqb, bdl_ref[...], preferred_element_type=f32) + bln_ref[...])


def _full(shape):
    ndim = len(shape)
    return pl.BlockSpec(shape, lambda i, *, _nd=ndim: (0,) * _nd)


def kernel(E, X_part, X_obj, nodes, obj_class, params):
    p = params
    f32 = jnp.float32

    def r2(v):
        return v.reshape(1, -1)

    XT = X_part.transpose(0, 2, 1)

    H2T = pl.pallas_call(
        _gcn_body,
        grid=(NG,),
        in_specs=[
            pl.BlockSpec((BBLK, N, N), lambda i: (i, 0, 0)),
            pl.BlockSpec((BBLK, FIN, N), lambda i: (i, 0, 0)),
            _full((H1, FIN)), _full((H1, 1)),
            _full((H2, H1)), _full((H2, 1)),
        ],
        out_specs=pl.BlockSpec((BBLK, H2, N), lambda i: (i, 0, 0)),
        out_shape=jax.ShapeDtypeStruct((B, H2, N), f32),
    )(E, XT, p['gc1_W'].T, p['gc1_b'].reshape(H1, 1),
      p['gc2_W'].T, p['gc2_b'].reshape(H2, 1))

    h2f = H2T.reshape(B, N * H2)
    encW = p['enc_h3_W']
    encWa = encW[: N * H2].reshape(N, H2, H3).transpose(1, 0, 2).reshape(N * H2, H3)

    r = jnp.arange(N * GDH)
    cb = jnp.arange(N * BBXD)
    bdb = (p['bbx_W'][r % GDH][:, cb % BBXD]
           * (r[:, None] // GDH == cb[None, :] // BBXD)).astype(jnp.bfloat16)
    cl = jnp.arange(N)
    bdl = (p['lbl_W'][r % GDH, 0][:, None]
           * (r[:, None] // GDH == cl[None, :])).astype(jnp.bfloat16)
    bbn = jnp.tile(p['bbx_b'], N).reshape(1, N * BBXD)
    bln = jnp.full((1, N), p['lbl_b'][0], f32)

    dd1W = p['dd1_W']
    gd1W = p['gd1_W']
    weights = [
        encWa, encW[N * H2 :], r2(p['enc_h3_b']),
        p['zmean_W'], r2(p['zmean_b']), p['zlogvar_W'], r2(p['zlogvar_b']),
        p['de1_W'], r2(p['de1_b']), p['de2_W'], r2(p['de2_b']), p['de3_W'], r2(p['de3_b']),
        p['objc_W'], r2(p['objc_b']), p['part_W'], r2(p['part_b']),
        dd1W[:HPC], dd1W[HPC : HPC + HOC], dd1W[HPC + HOC :], r2(p['dd1_b']),
        p['dd2_W'], r2(p['dd2_b']), p['dd3_W'], r2(p['dd3_b']),
        gd1W[:HPC], gd1W[HPC : HPC + HOC], gd1W[HPC + HOC : HPC + HOC + LAT],
        gd1W[HPC + HOC + LAT :], r2(p['gd1_b']),
        p['gd2_W'], r2(p['gd2_b']), bdb, bdl, bbn, bln,
    ]
    z_mean, z_logvar, x_obj_bbx, xbn, xl = pl.pallas_call(
        _mlp_body,
        grid=(B // BBLK2,),
        in_specs=[
            pl.BlockSpec((BBLK2, N * H2), lambda i: (i, 0)),
            pl.BlockSpec((BBLK2, NOC), lambda i: (i, 0)),
            pl.BlockSpec((BBLK2, BBXD), lambda i: (i, 0)),
            pl.BlockSpec((BBLK2, N), lambda i: (i, 0)),
        ] + [_full(w.shape) for w in weights],
        out_specs=[
            pl.BlockSpec((BBLK2, LAT), lambda i: (i, 0)),
            pl.BlockSpec((BBLK2, LAT), lambda i: (i, 0)),
            pl.BlockSpec((BBLK2, BBXD), lambda i: (i, 0)),
            pl.BlockSpec((BBLK2, N * BBXD), lambda i: (i, 0)),
            pl.BlockSpec((BBLK2, N), lambda i: (i, 0)),
        ],
        out_shape=[
            jax.ShapeDtypeStruct((B, LAT), f32),
            jax.ShapeDtypeStruct((B, LAT), f32),
            jax.ShapeDtypeStruct((B, BBXD), f32),
            jax.ShapeDtypeStruct((B, N * BBXD), f32),
            jax.ShapeDtypeStruct((B, N), f32),
        ],
    )(h2f, obj_class, X_obj, nodes, *weights)

    x_bbx = xbn.reshape(B, N, BBXD)
    x_lbl = xl.reshape(B, N, LBLD)
    return (x_bbx, x_obj_bbx, x_lbl, z_mean, z_logvar)


# speedup vs baseline: 1.6630x; 1.0460x over previous
import jax
import jax.numpy as jnp
from jax.experimental import pallas as pl

B = 1024
N = 256
FIN = 7
H1 = 32
H2 = 16
H3 = 128
LAT = 64
NOC = 16
HOC = 8
HPC = 8
GDH = 16
BBXD = 6
LBLD = 1

BBLK = 16
WAVE = 8
NG = B // BBLK
BBLK2 = 256


def _gcn_body(E_ref, Xt_ref, w1t_ref, b1_ref, w2t_ref, b2_ref, h2t_ref):
    f32 = jnp.float32
    bf16 = jnp.bfloat16
    w1t = w1t_ref[...]
    b1 = b1_ref[...]
    w2t = w2t_ref[...]
    b2 = b2_ref[...]
    cdims = (((1,), (1,)), ((), ()))
    for w in range(BBLK // WAVE):
        bs = range(w * WAVE, (w + 1) * WAVE)
        t1s = [jax.lax.dot_general(Xt_ref[b].astype(bf16),
                                   E_ref[b].astype(bf16), cdims,
                                   preferred_element_type=f32) for b in bs]
        h1s = [jnp.maximum(jnp.dot(w1t, t1t, preferred_element_type=f32)
                           + b1, 0.0).astype(bf16) for t1t in t1s]
        t2s = [jax.lax.dot_general(h1s[k], E_ref[b].astype(bf16), cdims,
                                   preferred_element_type=f32)
               for k, b in enumerate(bs)]
        for k, b in enumerate(bs):
            h2t_ref[b] = jnp.maximum(
                jnp.dot(w2t, t2s[k], preferred_element_type=f32) + b2, 0.0)


def _mlp_body(h2f_ref, oc_ref, xo_ref, nd_ref,
              encWa_ref, encWb_ref, encb_ref,
              zmW_ref, zmb_ref, zlW_ref, zlb_ref,
              de1W_ref, de1b_ref, de2W_ref, de2b_ref, de3W_ref, de3b_ref,
              objW_ref, objb_ref, partW_ref, partb_ref,
              dd1Wa_ref, dd1Wb_ref, dd1Wc_ref, dd1b_ref,
              dd2W_ref, dd2b_ref, dd3W_ref, dd3b_ref,
              gd1Wa_ref, gd1Wb_ref, gd1Wc_ref, gd1Wd_ref, gd1b_ref,
              gd2W_ref, gd2b_ref, bdb_ref, bdl_ref, bbn_ref, bln_ref,
              zm_ref, zl_ref, xob_ref, xbn_ref, xl_ref):
    f32 = jnp.float32
    bf16 = jnp.bfloat16
    h2f = h2f_ref[...]
    oc_raw = oc_ref[...]
    h3 = jnp.maximum(
        jnp.dot(h2f, encWa_ref[...], preferred_element_type=f32)
        + jnp.dot(oc_raw, encWb_ref[...], preferred_element_type=f32)
        + encb_ref[...], 0.0)
    z_mean = jnp.dot(h3, zmW_ref[...], preferred_element_type=f32) + zmb_ref[...]
    z_logvar = jnp.dot(h3, zlW_ref[...], preferred_element_type=f32) + zlb_ref[...]
    zm_ref[...] = z_mean
    zl_ref[...] = z_logvar
    lo = jnp.maximum(jnp.dot(xo_ref[...], de1W_ref[...], preferred_element_type=f32) + de1b_ref[...], 0.0)
    lo = jnp.maximum(jnp.dot(lo, de2W_ref[...], preferred_element_type=f32) + de2b_ref[...], 0.0)
    latent_obj = jnp.dot(lo, de3W_ref[...], preferred_element_type=f32) + de3b_ref[...]
    oc = jnp.dot(oc_raw, objW_ref[...], preferred_element_type=f32) + objb_ref[...]
    nd = jnp.dot(nd_ref[...], partW_ref[...], preferred_element_type=f32) + partb_ref[...]
    d = jnp.maximum(
        jnp.dot(nd, dd1Wa_ref[...], preferred_element_type=f32)
        + jnp.dot(oc, dd1Wb_ref[...], preferred_element_type=f32)
        + jnp.dot(latent_obj, dd1Wc_ref[...], preferred_element_type=f32)
        + dd1b_ref[...], 0.0)
    d = jnp.maximum(jnp.dot(d, dd2W_ref[...], preferred_element_type=f32) + dd2b_ref[...], 0.0)
    xob_ref[...] = jax.nn.sigmoid(jnp.dot(d, dd3W_ref[...], preferred_element_type=f32) + dd3b_ref[...])
    a1 = jnp.maximum(
        jnp.dot(nd, gd1Wa_ref[...], preferred_element_type=f32)
        + jnp.dot(oc, gd1Wb_ref[...], preferred_element_type=f32)
        + jnp.dot(latent_obj, gd1Wc_ref[...], preferred_element_type=f32)
        + jnp.dot(z_mean, gd1Wd_ref[...], preferred_element_type=f32)
        + gd1b_ref[...], 0.0)
    gq = jnp.maximum(jnp.dot(a1, gd2W_ref[...], preferred_element_type=f32) + gd2b_ref[...], 0.0)
    gqb = gq.astype(bf16)
    xbn_ref[...] = jax.nn.sigmoid(
        jnp.dot(gqb, bdb_ref[...], preferred_element_type=f32) + bbn_ref[...])
    xl_ref[...] = jax.nn.sigmoid(
        jnp.dot(gqb, bdl_ref[...], preferred_element_type=f32) + bln_ref[...])


def _full(shape):
    ndim = len(shape)
    return pl.BlockSpec(shape, lambda i, *, _nd=ndim: (0,) * _nd)


def kernel(E, X_part, X_obj, nodes, obj_class, params):
    p = params
    f32 = jnp.float32

    def r2(v):
        return v.reshape(1, -1)

    XT = X_part.transpose(0, 2, 1)

    H2T = pl.pallas_call(
        _gcn_body,
        grid=(NG,),
        in_specs=[
            pl.BlockSpec((BBLK, N, N), lambda i: (i, 0, 0)),
            pl.BlockSpec((BBLK, FIN, N), lambda i: (i, 0, 0)),
            _full((H1, FIN)), _full((H1, 1)),
            _full((H2, H1)), _full((H2, 1)),
        ],
        out_specs=pl.BlockSpec((BBLK, H2, N), lambda i: (i, 0, 0)),
        out_shape=jax.ShapeDtypeStruct((B, H2, N), f32),
    )(E, XT, p['gc1_W'].T, p['gc1_b'].reshape(H1, 1),
      p['gc2_W'].T, p['gc2_b'].reshape(H2, 1))

    h2f = H2T.reshape(B, N * H2)
    encW = p['enc_h3_W']
    encWa = encW[: N * H2].reshape(N, H2, H3).transpose(1, 0, 2).reshape(N * H2, H3)

    r = jnp.arange(N * GDH)
    cb = jnp.arange(N * BBXD)
    bdb = (p['bbx_W'][r % GDH][:, cb % BBXD]
           * (r[:, None] // GDH == cb[None, :] // BBXD)).astype(jnp.bfloat16)
    cl = jnp.arange(N)
    bdl = (p['lbl_W'][r % GDH, 0][:, None]
           * (r[:, None] // GDH == cl[None, :])).astype(jnp.bfloat16)
    bbn = jnp.tile(p['bbx_b'], N).reshape(1, N * BBXD)
    bln = jnp.full((1, N), p['lbl_b'][0], f32)

    dd1W = p['dd1_W']
    gd1W = p['gd1_W']
    weights = [
        encWa, encW[N * H2 :], r2(p['enc_h3_b']),
        p['zmean_W'], r2(p['zmean_b']), p['zlogvar_W'], r2(p['zlogvar_b']),
        p['de1_W'], r2(p['de1_b']), p['de2_W'], r2(p['de2_b']), p['de3_W'], r2(p['de3_b']),
        p['objc_W'], r2(p['objc_b']), p['part_W'], r2(p['part_b']),
        dd1W[:HPC], dd1W[HPC : HPC + HOC], dd1W[HPC + HOC :], r2(p['dd1_b']),
        p['dd2_W'], r2(p['dd2_b']), p['dd3_W'], r2(p['dd3_b']),
        gd1W[:HPC], gd1W[HPC : HPC + HOC], gd1W[HPC + HOC : HPC + HOC + LAT],
        gd1W[HPC + HOC + LAT :], r2(p['gd1_b']),
        p['gd2_W'], r2(p['gd2_b']), bdb, bdl, bbn, bln,
    ]
    z_mean, z_logvar, x_obj_bbx, xbn, xl = pl.pallas_call(
        _mlp_body,
        grid=(B // BBLK2,),
        in_specs=[
            pl.BlockSpec((BBLK2, N * H2), lambda i: (i, 0)),
            pl.BlockSpec((BBLK2, NOC), lambda i: (i, 0)),
            pl.BlockSpec((BBLK2, BBXD), lambda i: (i, 0)),
            pl.BlockSpec((BBLK2, N), lambda i: (i, 0)),
        ] + [_full(w.shape) for w in weights],
        out_specs=[
            pl.BlockSpec((BBLK2, LAT), lambda i: (i, 0)),
            pl.BlockSpec((BBLK2, LAT), lambda i: (i, 0)),
            pl.BlockSpec((BBLK2, BBXD), lambda i: (i, 0)),
            pl.BlockSpec((BBLK2, N * BBXD), lambda i: (i, 0)),
            pl.BlockSpec((BBLK2, N), lambda i: (i, 0)),
        ],
        out_shape=[
            jax.ShapeDtypeStruct((B, LAT), f32),
            jax.ShapeDtypeStruct((B, LAT), f32),
            jax.ShapeDtypeStruct((B, BBXD), f32),
            jax.ShapeDtypeStruct((B, N * BBXD), f32),
            jax.ShapeDtypeStruct((B, N), f32),
        ],
    )(h2f, obj_class, X_obj, nodes, *weights)

    x_bbx = xbn.reshape(B, N, BBXD)
    x_lbl = xl.reshape(B, N, LBLD)
    return (x_bbx, x_obj_bbx, x_lbl, z_mean, z_logvar)


# speedup vs baseline: 1.7529x; 1.0541x over previous
import jax
import jax.numpy as jnp
from jax.experimental import pallas as pl

B = 1024
N = 256
FIN = 7
H1 = 32
H2 = 16
H3 = 128
LAT = 64
NOC = 16
HOC = 8
HPC = 8
GDH = 16
BBXD = 6
LBLD = 1

BBLK = 32
WAVE = 8
NG = B // BBLK
BBLK2 = 256


def _gcn_body(E_ref, Xt_ref, w1t_ref, b1_ref, w2t_ref, b2_ref, h2t_ref):
    f32 = jnp.float32
    bf16 = jnp.bfloat16
    w1t = w1t_ref[...]
    b1 = b1_ref[...]
    w2t = w2t_ref[...]
    b2 = b2_ref[...]
    cdims = (((1,), (1,)), ((), ()))
    for w in range(BBLK // WAVE):
        bs = range(w * WAVE, (w + 1) * WAVE)
        t1s = [jax.lax.dot_general(Xt_ref[b].astype(bf16),
                                   E_ref[b].astype(bf16), cdims,
                                   preferred_element_type=f32) for b in bs]
        h1s = [jnp.maximum(jnp.dot(w1t, t1t, preferred_element_type=f32)
                           + b1, 0.0).astype(bf16) for t1t in t1s]
        t2s = [jax.lax.dot_general(h1s[k], E_ref[b].astype(bf16), cdims,
                                   preferred_element_type=f32)
               for k, b in enumerate(bs)]
        for k, b in enumerate(bs):
            h2t_ref[b] = jnp.maximum(
                jnp.dot(w2t, t2s[k], preferred_element_type=f32) + b2, 0.0)


def _mlp_body(h2f_ref, oc_ref, xo_ref, nd_ref,
              encWa_ref, encWb_ref, encb_ref,
              zmW_ref, zmb_ref, zlW_ref, zlb_ref,
              de1W_ref, de1b_ref, de2W_ref, de2b_ref, de3W_ref, de3b_ref,
              objW_ref, objb_ref, partW_ref, partb_ref,
              dd1Wa_ref, dd1Wb_ref, dd1Wc_ref, dd1b_ref,
              dd2W_ref, dd2b_ref, dd3W_ref, dd3b_ref,
              gd1Wa_ref, gd1Wb_ref, gd1Wc_ref, gd1Wd_ref, gd1b_ref,
              gd2W_ref, gd2b_ref, bdb_ref, bdl_ref, bbn_ref, bln_ref,
              zm_ref, zl_ref, xob_ref, xbn_ref, xl_ref):
    f32 = jnp.float32
    bf16 = jnp.bfloat16
    h2f = h2f_ref[...]
    oc_raw = oc_ref[...]
    h3 = jnp.maximum(
        jnp.dot(h2f, encWa_ref[...], preferred_element_type=f32)
        + jnp.dot(oc_raw, encWb_ref[...], preferred_element_type=f32)
        + encb_ref[...], 0.0)
    z_mean = jnp.dot(h3, zmW_ref[...], preferred_element_type=f32) + zmb_ref[...]
    z_logvar = jnp.dot(h3, zlW_ref[...], preferred_element_type=f32) + zlb_ref[...]
    zm_ref[...] = z_mean
    zl_ref[...] = z_logvar
    lo = jnp.maximum(jnp.dot(xo_ref[...], de1W_ref[...], preferred_element_type=f32) + de1b_ref[...], 0.0)
    lo = jnp.maximum(jnp.dot(lo, de2W_ref[...], preferred_element_type=f32) + de2b_ref[...], 0.0)
    latent_obj = jnp.dot(lo, de3W_ref[...], preferred_element_type=f32) + de3b_ref[...]
    oc = jnp.dot(oc_raw, objW_ref[...], preferred_element_type=f32) + objb_ref[...]
    nd = jnp.dot(nd_ref[...], partW_ref[...], preferred_element_type=f32) + partb_ref[...]
    d = jnp.maximum(
        jnp.dot(nd, dd1Wa_ref[...], preferred_element_type=f32)
        + jnp.dot(oc, dd1Wb_ref[...], preferred_element_type=f32)
        + jnp.dot(latent_obj, dd1Wc_ref[...], preferred_element_type=f32)
        + dd1b_ref[...], 0.0)
    d = jnp.maximum(jnp.dot(d, dd2W_ref[...], preferred_element_type=f32) + dd2b_ref[...], 0.0)
    xob_ref[...] = jax.nn.sigmoid(jnp.dot(d, dd3W_ref[...], preferred_element_type=f32) + dd3b_ref[...])
    a1 = jnp.maximum(
        jnp.dot(nd, gd1Wa_ref[...], preferred_element_type=f32)
        + jnp.dot(oc, gd1Wb_ref[...], preferred_element_type=f32)
        + jnp.dot(latent_obj, gd1Wc_ref[...], preferred_element_type=f32)
        + jnp.dot(z_mean, gd1Wd_ref[...], preferred_element_type=f32)
        + gd1b_ref[...], 0.0)
    gq = jnp.maximum(jnp.dot(a1, gd2W_ref[...], preferred_element_type=f32) + gd2b_ref[...], 0.0)
    gqb = gq.astype(bf16)
    xbn_ref[...] = jax.nn.sigmoid(
        jnp.dot(gqb, bdb_ref[...], preferred_element_type=f32) + bbn_ref[...])
    xl_ref[...] = jax.nn.sigmoid(
        jnp.dot(gqb, bdl_ref[...], preferred_element_type=f32) + bln_ref[...])


def _full(shape):
    ndim = len(shape)
    return pl.BlockSpec(shape, lambda i, *, _nd=ndim: (0,) * _nd)


def kernel(E, X_part, X_obj, nodes, obj_class, params):
    p = params
    f32 = jnp.float32

    def r2(v):
        return v.reshape(1, -1)

    XT = X_part.transpose(0, 2, 1)

    H2T = pl.pallas_call(
        _gcn_body,
        grid=(NG,),
        in_specs=[
            pl.BlockSpec((BBLK, N, N), lambda i: (i, 0, 0)),
            pl.BlockSpec((BBLK, FIN, N), lambda i: (i, 0, 0)),
            _full((H1, FIN)), _full((H1, 1)),
            _full((H2, H1)), _full((H2, 1)),
        ],
        out_specs=pl.BlockSpec((BBLK, H2, N), lambda i: (i, 0, 0)),
        out_shape=jax.ShapeDtypeStruct((B, H2, N), f32),
    )(E, XT, p['gc1_W'].T, p['gc1_b'].reshape(H1, 1),
      p['gc2_W'].T, p['gc2_b'].reshape(H2, 1))

    h2f = H2T.reshape(B, N * H2)
    encW = p['enc_h3_W']
    encWa = encW[: N * H2].reshape(N, H2, H3).transpose(1, 0, 2).reshape(N * H2, H3)

    r = jnp.arange(N * GDH)
    cb = jnp.arange(N * BBXD)
    bdb = (p['bbx_W'][r % GDH][:, cb % BBXD]
           * (r[:, None] // GDH == cb[None, :] // BBXD)).astype(jnp.bfloat16)
    cl = jnp.arange(N)
    bdl = (p['lbl_W'][r % GDH, 0][:, None]
           * (r[:, None] // GDH == cl[None, :])).astype(jnp.bfloat16)
    bbn = jnp.tile(p['bbx_b'], N).reshape(1, N * BBXD)
    bln = jnp.full((1, N), p['lbl_b'][0], f32)

    dd1W = p['dd1_W']
    gd1W = p['gd1_W']
    weights = [
        encWa, encW[N * H2 :], r2(p['enc_h3_b']),
        p['zmean_W'], r2(p['zmean_b']), p['zlogvar_W'], r2(p['zlogvar_b']),
        p['de1_W'], r2(p['de1_b']), p['de2_W'], r2(p['de2_b']), p['de3_W'], r2(p['de3_b']),
        p['objc_W'], r2(p['objc_b']), p['part_W'], r2(p['part_b']),
        dd1W[:HPC], dd1W[HPC : HPC + HOC], dd1W[HPC + HOC :], r2(p['dd1_b']),
        p['dd2_W'], r2(p['dd2_b']), p['dd3_W'], r2(p['dd3_b']),
        gd1W[:HPC], gd1W[HPC : HPC + HOC], gd1W[HPC + HOC : HPC + HOC + LAT],
        gd1W[HPC + HOC + LAT :], r2(p['gd1_b']),
        p['gd2_W'], r2(p['gd2_b']), bdb, bdl, bbn, bln,
    ]
    z_mean, z_logvar, x_obj_bbx, xbn, xl = pl.pallas_call(
        _mlp_body,
        grid=(B // BBLK2,),
        in_specs=[
            pl.BlockSpec((BBLK2, N * H2), lambda i: (i, 0)),
            pl.BlockSpec((BBLK2, NOC), lambda i: (i, 0)),
            pl.BlockSpec((BBLK2, BBXD), lambda i: (i, 0)),
            pl.BlockSpec((BBLK2, N), lambda i: (i, 0)),
        ] + [_full(w.shape) for w in weights],
        out_specs=[
            pl.BlockSpec((BBLK2, LAT), lambda i: (i, 0)),
            pl.BlockSpec((BBLK2, LAT), lambda i: (i, 0)),
            pl.BlockSpec((BBLK2, BBXD), lambda i: (i, 0)),
            pl.BlockSpec((BBLK2, N * BBXD), lambda i: (i, 0)),
            pl.BlockSpec((BBLK2, N), lambda i: (i, 0)),
        ],
        out_shape=[
            jax.ShapeDtypeStruct((B, LAT), f32),
            jax.ShapeDtypeStruct((B, LAT), f32),
            jax.ShapeDtypeStruct((B, BBXD), f32),
            jax.ShapeDtypeStruct((B, N * BBXD), f32),
            jax.ShapeDtypeStruct((B, N), f32),
        ],
    )(h2f, obj_class, X_obj, nodes, *weights)

    x_bbx = xbn.reshape(B, N, BBXD)
    x_lbl = xl.reshape(B, N, LBLD)
    return (x_bbx, x_obj_bbx, x_lbl, z_mean, z_logvar)


# speedup vs baseline: 1.7826x; 1.0169x over previous
import jax
import jax.numpy as jnp
from jax.experimental import pallas as pl

B = 1024
N = 256
FIN = 7
H1 = 32
H2 = 16
H3 = 128
LAT = 64
NOC = 16
HOC = 8
HPC = 8
GDH = 16
BBXD = 6
LBLD = 1

BBLK = 64
WAVE = 8
NG = B // BBLK
BBLK2 = 256


def _gcn_body(E_ref, Xt_ref, w1t_ref, b1_ref, w2t_ref, b2_ref, h2t_ref):
    f32 = jnp.float32
    bf16 = jnp.bfloat16
    w1t = w1t_ref[...]
    b1 = b1_ref[...]
    w2t = w2t_ref[...]
    b2 = b2_ref[...]
    cdims = (((1,), (1,)), ((), ()))
    for w in range(BBLK // WAVE):
        bs = range(w * WAVE, (w + 1) * WAVE)
        t1s = [jax.lax.dot_general(Xt_ref[b].astype(bf16),
                                   E_ref[b].astype(bf16), cdims,
                                   preferred_element_type=f32) for b in bs]
        h1s = [jnp.maximum(jnp.dot(w1t, t1t, preferred_element_type=f32)
                           + b1, 0.0).astype(bf16) for t1t in t1s]
        t2s = [jax.lax.dot_general(h1s[k], E_ref[b].astype(bf16), cdims,
                                   preferred_element_type=f32)
               for k, b in enumerate(bs)]
        for k, b in enumerate(bs):
            h2t_ref[b] = jnp.maximum(
                jnp.dot(w2t, t2s[k], preferred_element_type=f32) + b2, 0.0)


def _mlp_body(h2f_ref, oc_ref, xo_ref, nd_ref,
              encWa_ref, encWb_ref, encb_ref,
              zmW_ref, zmb_ref, zlW_ref, zlb_ref,
              de1W_ref, de1b_ref, de2W_ref, de2b_ref, de3W_ref, de3b_ref,
              objW_ref, objb_ref, partW_ref, partb_ref,
              dd1Wa_ref, dd1Wb_ref, dd1Wc_ref, dd1b_ref,
              dd2W_ref, dd2b_ref, dd3W_ref, dd3b_ref,
              gd1Wa_ref, gd1Wb_ref, gd1Wc_ref, gd1Wd_ref, gd1b_ref,
              gd2W_ref, gd2b_ref, bdb_ref, bdl_ref, bbn_ref, bln_ref,
              zm_ref, zl_ref, xob_ref, xbn_ref, xl_ref):
    f32 = jnp.float32
    bf16 = jnp.bfloat16
    h2f = h2f_ref[...]
    oc_raw = oc_ref[...]
    h3 = jnp.maximum(
        jnp.dot(h2f, encWa_ref[...], preferred_element_type=f32)
        + jnp.dot(oc_raw, encWb_ref[...], preferred_element_type=f32)
        + encb_ref[...], 0.0)
    z_mean = jnp.dot(h3, zmW_ref[...], preferred_element_type=f32) + zmb_ref[...]
    z_logvar = jnp.dot(h3, zlW_ref[...], preferred_element_type=f32) + zlb_ref[...]
    zm_ref[...] = z_mean
    zl_ref[...] = z_logvar
    lo = jnp.maximum(jnp.dot(xo_ref[...], de1W_ref[...], preferred_element_type=f32) + de1b_ref[...], 0.0)
    lo = jnp.maximum(jnp.dot(lo, de2W_ref[...], preferred_element_type=f32) + de2b_ref[...], 0.0)
    latent_obj = jnp.dot(lo, de3W_ref[...], preferred_element_type=f32) + de3b_ref[...]
    oc = jnp.dot(oc_raw, objW_ref[...], preferred_element_type=f32) + objb_ref[...]
    nd = jnp.dot(nd_ref[...], partW_ref[...], preferred_element_type=f32) + partb_ref[...]
    d = jnp.maximum(
        jnp.dot(nd, dd1Wa_ref[...], preferred_element_type=f32)
        + jnp.dot(oc, dd1Wb_ref[...], preferred_element_type=f32)
        + jnp.dot(latent_obj, dd1Wc_ref[...], preferred_element_type=f32)
        + dd1b_ref[...], 0.0)
    d = jnp.maximum(jnp.dot(d, dd2W_ref[...], preferred_element_type=f32) + dd2b_ref[...], 0.0)
    xob_ref[...] = jax.nn.sigmoid(jnp.dot(d, dd3W_ref[...], preferred_element_type=f32) + dd3b_ref[...])
    a1 = jnp.maximum(
        jnp.dot(nd, gd1Wa_ref[...], preferred_element_type=f32)
        + jnp.dot(oc, gd1Wb_ref[...], preferred_element_type=f32)
        + jnp.dot(latent_obj, gd1Wc_ref[...], preferred_element_type=f32)
        + jnp.dot(z_mean, gd1Wd_ref[...], preferred_element_type=f32)
        + gd1b_ref[...], 0.0)
    gq = jnp.maximum(jnp.dot(a1, gd2W_ref[...], preferred_element_type=f32) + gd2b_ref[...], 0.0)
    gqb = gq.astype(bf16)
    xbn_ref[...] = jax.nn.sigmoid(
        jnp.dot(gqb, bdb_ref[...], preferred_element_type=f32) + bbn_ref[...])
    xl_ref[...] = jax.nn.sigmoid(
        jnp.dot(gqb, bdl_ref[...], preferred_element_type=f32) + bln_ref[...])


def _full(shape):
    ndim = len(shape)
    return pl.BlockSpec(shape, lambda i, *, _nd=ndim: (0,) * _nd)


def kernel(E, X_part, X_obj, nodes, obj_class, params):
    p = params
    f32 = jnp.float32

    def r2(v):
        return v.reshape(1, -1)

    XT = X_part.transpose(0, 2, 1)

    H2T = pl.pallas_call(
        _gcn_body,
        grid=(NG,),
        in_specs=[
            pl.BlockSpec((BBLK, N, N), lambda i: (i, 0, 0)),
            pl.BlockSpec((BBLK, FIN, N), lambda i: (i, 0, 0)),
            _full((H1, FIN)), _full((H1, 1)),
            _full((H2, H1)), _full((H2, 1)),
        ],
        out_specs=pl.BlockSpec((BBLK, H2, N), lambda i: (i, 0, 0)),
        out_shape=jax.ShapeDtypeStruct((B, H2, N), f32),
    )(E, XT, p['gc1_W'].T, p['gc1_b'].reshape(H1, 1),
      p['gc2_W'].T, p['gc2_b'].reshape(H2, 1))

    h2f = H2T.reshape(B, N * H2)
    encW = p['enc_h3_W']
    encWa = encW[: N * H2].reshape(N, H2, H3).transpose(1, 0, 2).reshape(N * H2, H3)

    r = jnp.arange(N * GDH)
    cb = jnp.arange(N * BBXD)
    bdb = (p['bbx_W'][r % GDH][:, cb % BBXD]
           * (r[:, None] // GDH == cb[None, :] // BBXD)).astype(jnp.bfloat16)
    cl = jnp.arange(N)
    bdl = (p['lbl_W'][r % GDH, 0][:, None]
           * (r[:, None] // GDH == cl[None, :])).astype(jnp.bfloat16)
    bbn = jnp.tile(p['bbx_b'], N).reshape(1, N * BBXD)
    bln = jnp.full((1, N), p['lbl_b'][0], f32)

    dd1W = p['dd1_W']
    gd1W = p['gd1_W']
    weights = [
        encWa, encW[N * H2 :], r2(p['enc_h3_b']),
        p['zmean_W'], r2(p['zmean_b']), p['zlogvar_W'], r2(p['zlogvar_b']),
        p['de1_W'], r2(p['de1_b']), p['de2_W'], r2(p['de2_b']), p['de3_W'], r2(p['de3_b']),
        p['objc_W'], r2(p['objc_b']), p['part_W'], r2(p['part_b']),
        dd1W[:HPC], dd1W[HPC : HPC + HOC], dd1W[HPC + HOC :], r2(p['dd1_b']),
        p['dd2_W'], r2(p['dd2_b']), p['dd3_W'], r2(p['dd3_b']),
        gd1W[:HPC], gd1W[HPC : HPC + HOC], gd1W[HPC + HOC : HPC + HOC + LAT],
        gd1W[HPC + HOC + LAT :], r2(p['gd1_b']),
        p['gd2_W'], r2(p['gd2_b']), bdb, bdl, bbn, bln,
    ]
    z_mean, z_logvar, x_obj_bbx, xbn, xl = pl.pallas_call(
        _mlp_body,
        grid=(B // BBLK2,),
        in_specs=[
            pl.BlockSpec((BBLK2, N * H2), lambda i: (i, 0)),
            pl.BlockSpec((BBLK2, NOC), lambda i: (i, 0)),
            pl.BlockSpec((BBLK2, BBXD), lambda i: (i, 0)),
            pl.BlockSpec((BBLK2, N), lambda i: (i, 0)),
        ] + [_full(w.shape) for w in weights],
        out_specs=[
            pl.BlockSpec((BBLK2, LAT), lambda i: (i, 0)),
            pl.BlockSpec((BBLK2, LAT), lambda i: (i, 0)),
            pl.BlockSpec((BBLK2, BBXD), lambda i: (i, 0)),
            pl.BlockSpec((BBLK2, N * BBXD), lambda i: (i, 0)),
            pl.BlockSpec((BBLK2, N), lambda i: (i, 0)),
        ],
        out_shape=[
            jax.ShapeDtypeStruct((B, LAT), f32),
            jax.ShapeDtypeStruct((B, LAT), f32),
            jax.ShapeDtypeStruct((B, BBXD), f32),
            jax.ShapeDtypeStruct((B, N * BBXD), f32),
            jax.ShapeDtypeStruct((B, N), f32),
        ],
    )(h2f, obj_class, X_obj, nodes, *weights)

    x_bbx = xbn.reshape(B, N, BBXD)
    x_lbl = xl.reshape(B, N, LBLD)
    return (x_bbx, x_obj_bbx, x_lbl, z_mean, z_logvar)
